# baseline (speedup 1.0000x reference)
import functools

import jax
import jax.numpy as jnp
from jax.experimental import pallas as pl
from jax.experimental.pallas import tpu as pltpu

B, L, D = 1024, 200, 128
H, O = 512, 128
TOP_K = 50
TB = 8


def _pool_kernel(seq_ref, mask_ref, pos_ref, w1i_ref, b1i_ref, w2i_ref,
                 w1e_ref, b1e_ref, w2e_ref, b2e_ref, out_ref,
                 p_imp_scr, p_enc_scr):
    step = pl.program_id(0)

    @pl.when(step == 0)
    def _():
        pos = pos_ref[...]
        p_enc_scr[...] = (
            jnp.dot(pos, w1e_ref[D:, :], preferred_element_type=jnp.float32)
            + b1e_ref[...])

    del p_imp_scr
    seq = seq_ref[...]
    seq2d = seq.reshape(TB * L, D)

    pos_b = jnp.broadcast_to(pos_ref[...][None, :, :], (TB, L, D))
    comb2d = jnp.concatenate([seq, pos_b], axis=-1).reshape(TB * L, 2 * D)
    h = jnp.maximum(
        jnp.dot(comb2d, w1i_ref[...], preferred_element_type=jnp.float32)
        + b1i_ref[...], 0.0)
    scores = jnp.dot(h, w2i_ref[...],
                     preferred_element_type=jnp.float32).reshape(TB, L)
    scores = jnp.where(mask_ref[...] == 0, jnp.float32(-1e9), scores)

    s_i = scores[:, :, None]
    s_j = scores[:, None, :]
    ii = jax.lax.broadcasted_iota(jnp.int32, (TB, L, L), 1)
    jj = jax.lax.broadcasted_iota(jnp.int32, (TB, L, L), 2)
    beats = (s_j > s_i) | ((s_j == s_i) & (jj < ii))
    rank = jnp.sum(jnp.where(beats, 1.0, 0.0), axis=2)
    selw = (rank < TOP_K).astype(jnp.float32)

    a2 = jnp.dot(seq2d, w1e_ref[:D, :], preferred_element_type=jnp.float32)
    h2 = jnp.maximum(a2.reshape(TB, L, H) + p_enc_scr[...][None, :, :], 0.0)
    pooled = jnp.sum(h2 * selw[:, :, None], axis=1) * (1.0 / TOP_K)
    out_ref[...] = (
        jnp.dot(pooled, w2e_ref[...], preferred_element_type=jnp.float32)
        + b2e_ref[...])


@jax.jit
def kernel(sequence_emb, mask, pos_table, W1_imp, b1_imp, W2_imp, b2_imp,
           W1_enc, b1_enc, W2_enc, b2_enc):
    del b2_imp
    pos = pos_table[:L]
    b1i = b1_imp.reshape(1, H)
    b1e = b1_enc.reshape(1, H)
    w2i = W2_imp.reshape(H, 1)
    b2e = b2_enc.reshape(1, O)

    grid = (B // TB,)
    out = pl.pallas_call(
        _pool_kernel,
        grid=grid,
        in_specs=[
            pl.BlockSpec((TB, L, D), lambda i: (i, 0, 0)),
            pl.BlockSpec((TB, L), lambda i: (i, 0)),
            pl.BlockSpec((L, D), lambda i: (0, 0)),
            pl.BlockSpec((2 * D, H), lambda i: (0, 0)),
            pl.BlockSpec((1, H), lambda i: (0, 0)),
            pl.BlockSpec((H, 1), lambda i: (0, 0)),
            pl.BlockSpec((2 * D, H), lambda i: (0, 0)),
            pl.BlockSpec((1, H), lambda i: (0, 0)),
            pl.BlockSpec((H, O), lambda i: (0, 0)),
            pl.BlockSpec((1, O), lambda i: (0, 0)),
        ],
        out_specs=pl.BlockSpec((TB, O), lambda i: (i, 0)),
        out_shape=jax.ShapeDtypeStruct((B, O), jnp.float32),
        scratch_shapes=[
            pltpu.VMEM((L, H), jnp.float32),
            pltpu.VMEM((L, H), jnp.float32),
        ],
        compiler_params=pltpu.CompilerParams(
            dimension_semantics=("arbitrary",),
        ),
    )(sequence_emb, mask, pos, W1_imp, b1i, w2i, W1_enc, b1e, W2_enc, b2e)
    return out


# speedup vs baseline: 1.0430x; 1.0430x over previous
import functools

import jax
import jax.numpy as jnp
from jax.experimental import pallas as pl
from jax.experimental.pallas import tpu as pltpu

B, L, D = 1024, 200, 128
H, O = 512, 128
TOP_K = 50
TB = 8


def _pool_kernel(seq_ref, mask_ref, pos_ref, w1i_ref, b1i_ref, w2i_ref,
                 w1e_ref, b1e_ref, w2e_ref, b2e_ref, out_ref,
                 p_imp_scr, p_enc_scr):
    step = pl.program_id(0)

    @pl.when(step == 0)
    def _():
        pos = pos_ref[...]
        p_imp_scr[...] = jnp.dot(
            pos, w1i_ref[D:, :], preferred_element_type=jnp.float32)
        p_enc_scr[...] = (
            jnp.dot(pos, w1e_ref[D:, :], preferred_element_type=jnp.float32)
            + b1e_ref[...])

    seq = seq_ref[...]
    seq2d = seq.reshape(TB * L, D)

    a = jnp.dot(seq2d, w1i_ref[:D, :], preferred_element_type=jnp.float32)
    h = jnp.maximum(
        (a.reshape(TB, L, H) + p_imp_scr[...][None, :, :]).reshape(TB * L, H)
        + b1i_ref[...], 0.0)
    scores = jnp.dot(h, w2i_ref[...],
                     preferred_element_type=jnp.float32).reshape(TB, L)
    scores = jnp.where(mask_ref[...] == 0, jnp.float32(-1e9), scores)

    s_i = scores[:, :, None]
    s_j = scores[:, None, :]
    ii = jax.lax.broadcasted_iota(jnp.int32, (TB, L, L), 1)
    jj = jax.lax.broadcasted_iota(jnp.int32, (TB, L, L), 2)
    beats = (s_j > s_i) | ((s_j == s_i) & (jj < ii))
    rank = jnp.sum(jnp.where(beats, 1.0, 0.0), axis=2)
    selw = (rank < TOP_K).astype(jnp.float32)

    a2 = jnp.dot(seq2d, w1e_ref[:D, :], preferred_element_type=jnp.float32)
    h2 = jnp.maximum(a2.reshape(TB, L, H) + p_enc_scr[...][None, :, :], 0.0)
    pooled = jnp.sum(h2 * selw[:, :, None], axis=1) * (1.0 / TOP_K)
    out_ref[...] = (
        jnp.dot(pooled, w2e_ref[...], preferred_element_type=jnp.float32)
        + b2e_ref[...])


@jax.jit
def kernel(sequence_emb, mask, pos_table, W1_imp, b1_imp, W2_imp, b2_imp,
           W1_enc, b1_enc, W2_enc, b2_enc):
    del b2_imp
    pos = pos_table[:L]
    b1i = b1_imp.reshape(1, H)
    b1e = b1_enc.reshape(1, H)
    w2i = W2_imp.reshape(H, 1)
    b2e = b2_enc.reshape(1, O)

    grid = (B // TB,)
    out = pl.pallas_call(
        _pool_kernel,
        grid=grid,
        in_specs=[
            pl.BlockSpec((TB, L, D), lambda i: (i, 0, 0)),
            pl.BlockSpec((TB, L), lambda i: (i, 0)),
            pl.BlockSpec((L, D), lambda i: (0, 0)),
            pl.BlockSpec((2 * D, H), lambda i: (0, 0)),
            pl.BlockSpec((1, H), lambda i: (0, 0)),
            pl.BlockSpec((H, 1), lambda i: (0, 0)),
            pl.BlockSpec((2 * D, H), lambda i: (0, 0)),
            pl.BlockSpec((1, H), lambda i: (0, 0)),
            pl.BlockSpec((H, O), lambda i: (0, 0)),
            pl.BlockSpec((1, O), lambda i: (0, 0)),
        ],
        out_specs=pl.BlockSpec((TB, O), lambda i: (i, 0)),
        out_shape=jax.ShapeDtypeStruct((B, O), jnp.float32),
        scratch_shapes=[
            pltpu.VMEM((L, H), jnp.float32),
            pltpu.VMEM((L, H), jnp.float32),
        ],
        compiler_params=pltpu.CompilerParams(
            dimension_semantics=("arbitrary",),
        ),
    )(sequence_emb, mask, pos, W1_imp, b1i, w2i, W1_enc, b1e, W2_enc, b2e)
    return out


# speedup vs baseline: 1.1473x; 1.1000x over previous
import jax
import jax.numpy as jnp
from jax.experimental import pallas as pl
from jax.experimental.pallas import tpu as pltpu

B, L, D = 1024, 200, 128
H, O = 512, 128
TOP_K = 50
TB = 16


def _pos_kernel(pos_ref, w1i_ref, w1e_ref, b1e_ref, p_imp_ref, p_enc_ref):
    pos = pos_ref[...]
    p_imp_ref[...] = jnp.dot(
        pos, w1i_ref[D:, :], preferred_element_type=jnp.float32)
    p_enc_ref[...] = (
        jnp.dot(pos, w1e_ref[D:, :], preferred_element_type=jnp.float32)
        + b1e_ref[...])


def _pool_kernel(seq_ref, mask_ref, p_imp_ref, p_enc_ref, w1i_ref, b1i_ref,
                 w2i_ref, w1e_ref, w2e_ref, b2e_ref, out_ref):
    seq = seq_ref[...]
    seq2d = seq.reshape(TB * L, D)

    a = jnp.dot(seq2d, w1i_ref[:D, :], preferred_element_type=jnp.float32)
    h = jnp.maximum(
        (a.reshape(TB, L, H) + p_imp_ref[...][None, :, :]).reshape(TB * L, H)
        + b1i_ref[...], 0.0)
    scores = jnp.dot(h, w2i_ref[...],
                     preferred_element_type=jnp.float32).reshape(TB, L)
    scores = jnp.where(mask_ref[...] == 0, jnp.float32(-1e9), scores)

    s_i = scores[:, :, None]
    s_j = scores[:, None, :]
    ii = jax.lax.broadcasted_iota(jnp.int32, (TB, L, L), 1)
    jj = jax.lax.broadcasted_iota(jnp.int32, (TB, L, L), 2)
    beats = (s_j > s_i) | ((s_j == s_i) & (jj < ii))
    rank = jnp.sum(jnp.where(beats, 1.0, 0.0), axis=2)
    selw = (rank < TOP_K).astype(jnp.float32)

    a2 = jnp.dot(seq2d, w1e_ref[:D, :], preferred_element_type=jnp.float32)
    h2 = jnp.maximum(a2.reshape(TB, L, H) + p_enc_ref[...][None, :, :], 0.0)
    pooled = jnp.sum(h2 * selw[:, :, None], axis=1) * (1.0 / TOP_K)
    out_ref[...] = (
        jnp.dot(pooled, w2e_ref[...], preferred_element_type=jnp.float32)
        + b2e_ref[...])


@jax.jit
def kernel(sequence_emb, mask, pos_table, W1_imp, b1_imp, W2_imp, b2_imp,
           W1_enc, b1_enc, W2_enc, b2_enc):
    del b2_imp
    pos = pos_table[:L]
    b1i = b1_imp.reshape(1, H)
    b1e = b1_enc.reshape(1, H)
    w2i = W2_imp.reshape(H, 1)
    b2e = b2_enc.reshape(1, O)

    p_imp, p_enc = pl.pallas_call(
        _pos_kernel,
        in_specs=[
            pl.BlockSpec((L, D), lambda: (0, 0)),
            pl.BlockSpec((2 * D, H), lambda: (0, 0)),
            pl.BlockSpec((2 * D, H), lambda: (0, 0)),
            pl.BlockSpec((1, H), lambda: (0, 0)),
        ],
        out_specs=[
            pl.BlockSpec((L, H), lambda: (0, 0)),
            pl.BlockSpec((L, H), lambda: (0, 0)),
        ],
        out_shape=[
            jax.ShapeDtypeStruct((L, H), jnp.float32),
            jax.ShapeDtypeStruct((L, H), jnp.float32),
        ],
    )(pos, W1_imp, W1_enc, b1e)

    grid = (B // TB,)
    out = pl.pallas_call(
        _pool_kernel,
        grid=grid,
        in_specs=[
            pl.BlockSpec((TB, L, D), lambda i: (i, 0, 0)),
            pl.BlockSpec((TB, L), lambda i: (i, 0)),
            pl.BlockSpec((L, H), lambda i: (0, 0)),
            pl.BlockSpec((L, H), lambda i: (0, 0)),
            pl.BlockSpec((2 * D, H), lambda i: (0, 0)),
            pl.BlockSpec((1, H), lambda i: (0, 0)),
            pl.BlockSpec((H, 1), lambda i: (0, 0)),
            pl.BlockSpec((2 * D, H), lambda i: (0, 0)),
            pl.BlockSpec((H, O), lambda i: (0, 0)),
            pl.BlockSpec((1, O), lambda i: (0, 0)),
        ],
        out_specs=pl.BlockSpec((TB, O), lambda i: (i, 0)),
        out_shape=jax.ShapeDtypeStruct((B, O), jnp.float32),
        compiler_params=pltpu.CompilerParams(
            dimension_semantics=("parallel",),
        ),
    )(sequence_emb, mask, p_imp, p_enc, W1_imp, b1i, w2i, W1_enc, W2_enc, b2e)
    return out


# speedup vs baseline: 1.1687x; 1.0186x over previous
import jax
import jax.numpy as jnp
from jax.experimental import pallas as pl
from jax.experimental.pallas import tpu as pltpu

B, L, D = 1024, 200, 128
H, O = 512, 128
TOP_K = 50
TB = 32


def _pos_kernel(pos_ref, w1i_ref, w1e_ref, b1e_ref, p_imp_ref, p_enc_ref):
    pos = pos_ref[...]
    p_imp_ref[...] = jnp.dot(
        pos, w1i_ref[D:, :], preferred_element_type=jnp.float32)
    p_enc_ref[...] = (
        jnp.dot(pos, w1e_ref[D:, :], preferred_element_type=jnp.float32)
        + b1e_ref[...])


def _pool_kernel(seq_ref, mask_ref, p_imp_ref, p_enc_ref, w1i_ref, b1i_ref,
                 w2i_ref, w1e_ref, w2e_ref, b2e_ref, out_ref):
    seq = seq_ref[...]
    seq2d = seq.reshape(TB * L, D)

    a = jnp.dot(seq2d, w1i_ref[:D, :], preferred_element_type=jnp.float32)
    h = jnp.maximum(
        (a.reshape(TB, L, H) + p_imp_ref[...][None, :, :]).reshape(TB * L, H)
        + b1i_ref[...], 0.0)
    scores = jnp.dot(h, w2i_ref[...],
                     preferred_element_type=jnp.float32).reshape(TB, L)
    scores = jnp.where(mask_ref[...] == 0, jnp.float32(-1e9), scores)

    s_i = scores[:, :, None]
    s_j = scores[:, None, :]
    ii = jax.lax.broadcasted_iota(jnp.int32, (TB, L, L), 1)
    jj = jax.lax.broadcasted_iota(jnp.int32, (TB, L, L), 2)
    beats = (s_j > s_i) | ((s_j == s_i) & (jj < ii))
    rank = jnp.sum(jnp.where(beats, 1.0, 0.0), axis=2)
    selw = (rank < TOP_K).astype(jnp.float32)

    a2 = jnp.dot(seq2d, w1e_ref[:D, :], preferred_element_type=jnp.float32)
    h2 = jnp.maximum(a2.reshape(TB, L, H) + p_enc_ref[...][None, :, :], 0.0)
    pooled = jnp.sum(h2 * selw[:, :, None], axis=1) * (1.0 / TOP_K)
    out_ref[...] = (
        jnp.dot(pooled, w2e_ref[...], preferred_element_type=jnp.float32)
        + b2e_ref[...])


@jax.jit
def kernel(sequence_emb, mask, pos_table, W1_imp, b1_imp, W2_imp, b2_imp,
           W1_enc, b1_enc, W2_enc, b2_enc):
    del b2_imp
    pos = pos_table[:L]
    b1i = b1_imp.reshape(1, H)
    b1e = b1_enc.reshape(1, H)
    w2i = W2_imp.reshape(H, 1)
    b2e = b2_enc.reshape(1, O)

    p_imp, p_enc = pl.pallas_call(
        _pos_kernel,
        in_specs=[
            pl.BlockSpec((L, D), lambda: (0, 0)),
            pl.BlockSpec((2 * D, H), lambda: (0, 0)),
            pl.BlockSpec((2 * D, H), lambda: (0, 0)),
            pl.BlockSpec((1, H), lambda: (0, 0)),
        ],
        out_specs=[
            pl.BlockSpec((L, H), lambda: (0, 0)),
            pl.BlockSpec((L, H), lambda: (0, 0)),
        ],
        out_shape=[
            jax.ShapeDtypeStruct((L, H), jnp.float32),
            jax.ShapeDtypeStruct((L, H), jnp.float32),
        ],
    )(pos, W1_imp, W1_enc, b1e)

    grid = (B // TB,)
    out = pl.pallas_call(
        _pool_kernel,
        grid=grid,
        in_specs=[
            pl.BlockSpec((TB, L, D), lambda i: (i, 0, 0)),
            pl.BlockSpec((TB, L), lambda i: (i, 0)),
            pl.BlockSpec((L, H), lambda i: (0, 0)),
            pl.BlockSpec((L, H), lambda i: (0, 0)),
            pl.BlockSpec((2 * D, H), lambda i: (0, 0)),
            pl.BlockSpec((1, H), lambda i: (0, 0)),
            pl.BlockSpec((H, 1), lambda i: (0, 0)),
            pl.BlockSpec((2 * D, H), lambda i: (0, 0)),
            pl.BlockSpec((H, O), lambda i: (0, 0)),
            pl.BlockSpec((1, O), lambda i: (0, 0)),
        ],
        out_specs=pl.BlockSpec((TB, O), lambda i: (i, 0)),
        out_shape=jax.ShapeDtypeStruct((B, O), jnp.float32),
        compiler_params=pltpu.CompilerParams(
            dimension_semantics=("parallel",),
        ),
    )(sequence_emb, mask, p_imp, p_enc, W1_imp, b1i, w2i, W1_enc, W2_enc, b2e)
    return out


# speedup vs baseline: 1.1868x; 1.0154x over previous
import jax
import jax.numpy as jnp
from jax.experimental import pallas as pl
from jax.experimental.pallas import tpu as pltpu

B, L, D = 1024, 200, 128
H, O = 512, 128
TOP_K = 50
TB = 32


def _pos_kernel(pos_ref, w1i_ref, w1e_ref, b1e_ref, p_imp_ref, p_enc_ref):
    pos = pos_ref[...]
    p_imp_ref[...] = jnp.dot(
        pos, w1i_ref[D:, :], preferred_element_type=jnp.float32)
    p_enc_ref[...] = (
        jnp.dot(pos, w1e_ref[D:, :], preferred_element_type=jnp.float32)
        + b1e_ref[...])


def _pool_kernel(seq_ref, mask_ref, p_imp_ref, p_enc_ref, w1i_ref, b1i_ref,
                 w2i_ref, w1e_ref, w2e_ref, b2e_ref, out_ref):
    seq = seq_ref[...]
    seq2d = seq.reshape(TB * L, D)
    dn = (((2,), (0,)), ((), ()))

    a2 = jax.lax.dot_general(seq, w1e_ref[:D, :], dn,
                             preferred_element_type=jnp.float32)

    a = jax.lax.dot_general(seq, w1i_ref[:D, :], dn,
                            preferred_element_type=jnp.float32)
    h = jnp.maximum(a + (p_imp_ref[...] + b1i_ref[...])[None, :, :], 0.0)
    scores = jnp.dot(h.reshape(TB * L, H), w2i_ref[...],
                     preferred_element_type=jnp.float32).reshape(TB, L)
    scores = jnp.where(mask_ref[...] == 0, jnp.float32(-1e9), scores)

    s_i = scores[:, :, None]
    s_j = scores[:, None, :]
    ii = jax.lax.broadcasted_iota(jnp.int32, (TB, L, L), 1)
    jj = jax.lax.broadcasted_iota(jnp.int32, (TB, L, L), 2)
    beats = (s_j > s_i) | ((s_j == s_i) & (jj < ii))
    rank = jnp.sum(jnp.where(beats, 1.0, 0.0), axis=2)
    selw = (rank < TOP_K).astype(jnp.float32)

    del seq2d
    h2 = jnp.maximum(a2 + p_enc_ref[...][None, :, :], 0.0)
    pooled = jnp.sum(h2 * selw[:, :, None], axis=1) * (1.0 / TOP_K)
    out_ref[...] = (
        jnp.dot(pooled, w2e_ref[...], preferred_element_type=jnp.float32)
        + b2e_ref[...])


@jax.jit
def kernel(sequence_emb, mask, pos_table, W1_imp, b1_imp, W2_imp, b2_imp,
           W1_enc, b1_enc, W2_enc, b2_enc):
    del b2_imp
    pos = pos_table[:L]
    b1i = b1_imp.reshape(1, H)
    b1e = b1_enc.reshape(1, H)
    w2i = W2_imp.reshape(H, 1)
    b2e = b2_enc.reshape(1, O)

    p_imp, p_enc = pl.pallas_call(
        _pos_kernel,
        in_specs=[
            pl.BlockSpec((L, D), lambda: (0, 0)),
            pl.BlockSpec((2 * D, H), lambda: (0, 0)),
            pl.BlockSpec((2 * D, H), lambda: (0, 0)),
            pl.BlockSpec((1, H), lambda: (0, 0)),
        ],
        out_specs=[
            pl.BlockSpec((L, H), lambda: (0, 0)),
            pl.BlockSpec((L, H), lambda: (0, 0)),
        ],
        out_shape=[
            jax.ShapeDtypeStruct((L, H), jnp.float32),
            jax.ShapeDtypeStruct((L, H), jnp.float32),
        ],
    )(pos, W1_imp, W1_enc, b1e)

    grid = (B // TB,)
    out = pl.pallas_call(
        _pool_kernel,
        grid=grid,
        in_specs=[
            pl.BlockSpec((TB, L, D), lambda i: (i, 0, 0)),
            pl.BlockSpec((TB, L), lambda i: (i, 0)),
            pl.BlockSpec((L, H), lambda i: (0, 0)),
            pl.BlockSpec((L, H), lambda i: (0, 0)),
            pl.BlockSpec((2 * D, H), lambda i: (0, 0)),
            pl.BlockSpec((1, H), lambda i: (0, 0)),
            pl.BlockSpec((H, 1), lambda i: (0, 0)),
            pl.BlockSpec((2 * D, H), lambda i: (0, 0)),
            pl.BlockSpec((H, O), lambda i: (0, 0)),
            pl.BlockSpec((1, O), lambda i: (0, 0)),
        ],
        out_specs=pl.BlockSpec((TB, O), lambda i: (i, 0)),
        out_shape=jax.ShapeDtypeStruct((B, O), jnp.float32),
        compiler_params=pltpu.CompilerParams(
            dimension_semantics=("parallel",),
        ),
    )(sequence_emb, mask, p_imp, p_enc, W1_imp, b1i, w2i, W1_enc, W2_enc, b2e)
    return out


# speedup vs baseline: 1.5011x; 1.2649x over previous
import jax
import jax.numpy as jnp
from jax.experimental import pallas as pl
from jax.experimental.pallas import tpu as pltpu

B, L, D = 1024, 200, 128
H, O = 512, 128
TOP_K = 50
TB = 32
TH = 16


def _pos_kernel(pos_ref, w1i_ref, w1e_ref, b1e_ref, p_imp_ref, p_enc_ref):
    pos = pos_ref[...]
    p_imp_ref[...] = jnp.dot(
        pos, w1i_ref[D:, :], preferred_element_type=jnp.float32)
    p_enc_ref[...] = (
        jnp.dot(pos, w1e_ref[D:, :], preferred_element_type=jnp.float32)
        + b1e_ref[...])


def _pool_kernel(seq_ref, mask_ref, p_imp_ref, p_enc_ref, w1i_ref, b1i_ref,
                 w2i_ref, w1e_ref, w2e_ref, b2e_ref, out_ref):
    dn = (((2,), (0,)), ((), ()))
    p_imp = (p_imp_ref[...] + b1i_ref[...])[None, :, :]
    p_enc = p_enc_ref[...][None, :, :]

    def hidden(half):
        seq = seq_ref[pl.ds(half * TH, TH), :, :]
        a = jax.lax.dot_general(seq, w1i_ref[:D, :], dn,
                                preferred_element_type=jnp.float32)
        a2 = jax.lax.dot_general(seq, w1e_ref[:D, :], dn,
                                 preferred_element_type=jnp.float32)
        h = jnp.maximum(a + p_imp, 0.0)
        return h, a2

    def score(h, half):
        s = jnp.dot(h.reshape(TH * L, H), w2i_ref[...],
                    preferred_element_type=jnp.float32).reshape(TH, L)
        m = mask_ref[pl.ds(half * TH, TH), :]
        return jnp.where(m == 0, jnp.float32(-1e9), s)

    def pool(scores, a2, half):
        s_i = scores[:, :, None]
        s_j = scores[:, None, :]
        ii = jax.lax.broadcasted_iota(jnp.int32, (TH, L, L), 1)
        jj = jax.lax.broadcasted_iota(jnp.int32, (TH, L, L), 2)
        beats = (s_j > s_i) | ((s_j == s_i) & (jj < ii))
        rank = jnp.sum(jnp.where(beats, 1.0, 0.0), axis=2)
        selw = (rank < TOP_K).astype(jnp.float32)

        h2 = jnp.maximum(a2 + p_enc, 0.0)
        pooled = jnp.sum(h2 * selw[:, :, None], axis=1) * (1.0 / TOP_K)
        out_ref[pl.ds(half * TH, TH), :] = (
            jnp.dot(pooled, w2e_ref[...], preferred_element_type=jnp.float32)
            + b2e_ref[...])

    h0, a2_0 = hidden(0)
    s0 = score(h0, 0)
    h1, a2_1 = hidden(1)
    s1 = score(h1, 1)
    pool(s0, a2_0, 0)
    pool(s1, a2_1, 1)


@jax.jit
def kernel(sequence_emb, mask, pos_table, W1_imp, b1_imp, W2_imp, b2_imp,
           W1_enc, b1_enc, W2_enc, b2_enc):
    del b2_imp
    pos = pos_table[:L]
    b1i = b1_imp.reshape(1, H)
    b1e = b1_enc.reshape(1, H)
    w2i = W2_imp.reshape(H, 1)
    b2e = b2_enc.reshape(1, O)

    p_imp, p_enc = pl.pallas_call(
        _pos_kernel,
        in_specs=[
            pl.BlockSpec((L, D), lambda: (0, 0)),
            pl.BlockSpec((2 * D, H), lambda: (0, 0)),
            pl.BlockSpec((2 * D, H), lambda: (0, 0)),
            pl.BlockSpec((1, H), lambda: (0, 0)),
        ],
        out_specs=[
            pl.BlockSpec((L, H), lambda: (0, 0)),
            pl.BlockSpec((L, H), lambda: (0, 0)),
        ],
        out_shape=[
            jax.ShapeDtypeStruct((L, H), jnp.float32),
            jax.ShapeDtypeStruct((L, H), jnp.float32),
        ],
    )(pos, W1_imp, W1_enc, b1e)

    grid = (B // TB,)
    out = pl.pallas_call(
        _pool_kernel,
        grid=grid,
        in_specs=[
            pl.BlockSpec((TB, L, D), lambda i: (i, 0, 0)),
            pl.BlockSpec((TB, L), lambda i: (i, 0)),
            pl.BlockSpec((L, H), lambda i: (0, 0)),
            pl.BlockSpec((L, H), lambda i: (0, 0)),
            pl.BlockSpec((2 * D, H), lambda i: (0, 0)),
            pl.BlockSpec((1, H), lambda i: (0, 0)),
            pl.BlockSpec((H, 1), lambda i: (0, 0)),
            pl.BlockSpec((2 * D, H), lambda i: (0, 0)),
            pl.BlockSpec((H, O), lambda i: (0, 0)),
            pl.BlockSpec((1, O), lambda i: (0, 0)),
        ],
        out_specs=pl.BlockSpec((TB, O), lambda i: (i, 0)),
        out_shape=jax.ShapeDtypeStruct((B, O), jnp.float32),
        compiler_params=pltpu.CompilerParams(
            dimension_semantics=("parallel",),
        ),
    )(sequence_emb, mask, p_imp, p_enc, W1_imp, b1i, w2i, W1_enc, W2_enc, b2e)
    return out


# speedup vs baseline: 1.5573x; 1.0374x over previous
import jax
import jax.numpy as jnp
from jax.experimental import pallas as pl
from jax.experimental.pallas import tpu as pltpu

B, L, D = 1024, 200, 128
H, O = 512, 128
TOP_K = 50
TB = 32
TH = 8
NSUB = TB // TH


def _pos_kernel(pos_ref, w1i_ref, w1e_ref, b1e_ref, p_imp_ref, p_enc_ref):
    pos = pos_ref[...]
    p_imp_ref[...] = jnp.dot(
        pos, w1i_ref[D:, :], preferred_element_type=jnp.float32)
    p_enc_ref[...] = (
        jnp.dot(pos, w1e_ref[D:, :], preferred_element_type=jnp.float32)
        + b1e_ref[...])


def _pool_kernel(seq_ref, mask_ref, p_imp_ref, p_enc_ref, w1i_ref, b1i_ref,
                 w2i_ref, w1e_ref, w2e_ref, b2e_ref, out_ref):
    dn = (((2,), (0,)), ((), ()))
    p_imp = (p_imp_ref[...] + b1i_ref[...])[None, :, :]
    p_enc = p_enc_ref[...][None, :, :]

    def hidden(half):
        seq = seq_ref[pl.ds(half * TH, TH), :, :]
        a = jax.lax.dot_general(seq, w1i_ref[:D, :], dn,
                                preferred_element_type=jnp.float32)
        a2 = jax.lax.dot_general(seq, w1e_ref[:D, :], dn,
                                 preferred_element_type=jnp.float32)
        h = jnp.maximum(a + p_imp, 0.0)
        return h, a2

    def score(h, half):
        s = jnp.dot(h.reshape(TH * L, H), w2i_ref[...],
                    preferred_element_type=jnp.float32).reshape(TH, L)
        m = mask_ref[pl.ds(half * TH, TH), :]
        return jnp.where(m == 0, jnp.float32(-1e9), s)

    def pool(scores, a2, half):
        s_i = scores[:, :, None]
        s_j = scores[:, None, :]
        ii = jax.lax.broadcasted_iota(jnp.int32, (TH, L, L), 1)
        jj = jax.lax.broadcasted_iota(jnp.int32, (TH, L, L), 2)
        beats = (s_j > s_i) | ((s_j == s_i) & (jj < ii))
        rank = jnp.sum(jnp.where(beats, 1.0, 0.0), axis=2)
        selw = (rank < TOP_K).astype(jnp.float32)

        h2 = jnp.maximum(a2 + p_enc, 0.0)
        pooled = jnp.sum(h2 * selw[:, :, None], axis=1) * (1.0 / TOP_K)
        out_ref[pl.ds(half * TH, TH), :] = (
            jnp.dot(pooled, w2e_ref[...], preferred_element_type=jnp.float32)
            + b2e_ref[...])

    h_prev, a2_prev = hidden(0)
    s_prev = score(h_prev, 0)
    for half in range(1, NSUB):
        h_cur, a2_cur = hidden(half)
        s_cur = score(h_cur, half)
        pool(s_prev, a2_prev, half - 1)
        s_prev, a2_prev = s_cur, a2_cur
    pool(s_prev, a2_prev, NSUB - 1)


@jax.jit
def kernel(sequence_emb, mask, pos_table, W1_imp, b1_imp, W2_imp, b2_imp,
           W1_enc, b1_enc, W2_enc, b2_enc):
    del b2_imp
    pos = pos_table[:L]
    b1i = b1_imp.reshape(1, H)
    b1e = b1_enc.reshape(1, H)
    w2i = W2_imp.reshape(H, 1)
    b2e = b2_enc.reshape(1, O)

    p_imp, p_enc = pl.pallas_call(
        _pos_kernel,
        in_specs=[
            pl.BlockSpec((L, D), lambda: (0, 0)),
            pl.BlockSpec((2 * D, H), lambda: (0, 0)),
            pl.BlockSpec((2 * D, H), lambda: (0, 0)),
            pl.BlockSpec((1, H), lambda: (0, 0)),
        ],
        out_specs=[
            pl.BlockSpec((L, H), lambda: (0, 0)),
            pl.BlockSpec((L, H), lambda: (0, 0)),
        ],
        out_shape=[
            jax.ShapeDtypeStruct((L, H), jnp.float32),
            jax.ShapeDtypeStruct((L, H), jnp.float32),
        ],
    )(pos, W1_imp, W1_enc, b1e)

    grid = (B // TB,)
    out = pl.pallas_call(
        _pool_kernel,
        grid=grid,
        in_specs=[
            pl.BlockSpec((TB, L, D), lambda i: (i, 0, 0)),
            pl.BlockSpec((TB, L), lambda i: (i, 0)),
            pl.BlockSpec((L, H), lambda i: (0, 0)),
            pl.BlockSpec((L, H), lambda i: (0, 0)),
            pl.BlockSpec((2 * D, H), lambda i: (0, 0)),
            pl.BlockSpec((1, H), lambda i: (0, 0)),
            pl.BlockSpec((H, 1), lambda i: (0, 0)),
            pl.BlockSpec((2 * D, H), lambda i: (0, 0)),
            pl.BlockSpec((H, O), lambda i: (0, 0)),
            pl.BlockSpec((1, O), lambda i: (0, 0)),
        ],
        out_specs=pl.BlockSpec((TB, O), lambda i: (i, 0)),
        out_shape=jax.ShapeDtypeStruct((B, O), jnp.float32),
        compiler_params=pltpu.CompilerParams(
            dimension_semantics=("parallel",),
        ),
    )(sequence_emb, mask, p_imp, p_enc, W1_imp, b1i, w2i, W1_enc, W2_enc, b2e)
    return out


# speedup vs baseline: 1.5977x; 1.0260x over previous
import jax
import jax.numpy as jnp
from jax.experimental import pallas as pl
from jax.experimental.pallas import tpu as pltpu

B, L, D = 1024, 200, 128
H, O = 512, 128
TOP_K = 50
TB = 64
TH = 8
NSUB = TB // TH


def _pos_kernel(pos_ref, w1i_ref, w1e_ref, b1e_ref, p_imp_ref, p_enc_ref):
    pos = pos_ref[...]
    p_imp_ref[...] = jnp.dot(
        pos, w1i_ref[D:, :], preferred_element_type=jnp.float32)
    p_enc_ref[...] = (
        jnp.dot(pos, w1e_ref[D:, :], preferred_element_type=jnp.float32)
        + b1e_ref[...])


def _pool_kernel(seq_ref, mask_ref, p_imp_ref, p_enc_ref, w1i_ref, b1i_ref,
                 w2i_ref, w1e_ref, w2e_ref, b2e_ref, out_ref):
    dn = (((2,), (0,)), ((), ()))
    p_imp = (p_imp_ref[...] + b1i_ref[...])[None, :, :]
    p_enc = p_enc_ref[...][None, :, :]

    def hidden(half):
        seq = seq_ref[pl.ds(half * TH, TH), :, :]
        a = jax.lax.dot_general(seq, w1i_ref[:D, :], dn,
                                preferred_element_type=jnp.float32)
        a2 = jax.lax.dot_general(seq, w1e_ref[:D, :], dn,
                                 preferred_element_type=jnp.float32)
        h = jnp.maximum(a + p_imp, 0.0)
        return h, a2

    def score(h, half):
        s = jnp.dot(h.reshape(TH * L, H), w2i_ref[...],
                    preferred_element_type=jnp.float32).reshape(TH, L)
        m = mask_ref[pl.ds(half * TH, TH), :]
        return jnp.where(m == 0, jnp.float32(-1e9), s)

    def pool(scores, a2, half):
        s_i = scores[:, :, None]
        s_j = scores[:, None, :]
        ii = jax.lax.broadcasted_iota(jnp.int32, (TH, L, L), 1)
        jj = jax.lax.broadcasted_iota(jnp.int32, (TH, L, L), 2)
        beats = (s_j > s_i) | ((s_j == s_i) & (jj < ii))
        rank = jnp.sum(jnp.where(beats, 1.0, 0.0), axis=2)
        selw = (rank < TOP_K).astype(jnp.float32)

        h2 = jnp.maximum(a2 + p_enc, 0.0)
        pooled = jnp.sum(h2 * selw[:, :, None], axis=1) * (1.0 / TOP_K)
        out_ref[pl.ds(half * TH, TH), :] = (
            jnp.dot(pooled, w2e_ref[...], preferred_element_type=jnp.float32)
            + b2e_ref[...])

    h_prev, a2_prev = hidden(0)
    s_prev = score(h_prev, 0)
    for half in range(1, NSUB):
        h_cur, a2_cur = hidden(half)
        s_cur = score(h_cur, half)
        pool(s_prev, a2_prev, half - 1)
        s_prev, a2_prev = s_cur, a2_cur
    pool(s_prev, a2_prev, NSUB - 1)


@jax.jit
def kernel(sequence_emb, mask, pos_table, W1_imp, b1_imp, W2_imp, b2_imp,
           W1_enc, b1_enc, W2_enc, b2_enc):
    del b2_imp
    pos = pos_table[:L]
    b1i = b1_imp.reshape(1, H)
    b1e = b1_enc.reshape(1, H)
    w2i = W2_imp.reshape(H, 1)
    b2e = b2_enc.reshape(1, O)

    p_imp, p_enc = pl.pallas_call(
        _pos_kernel,
        in_specs=[
            pl.BlockSpec((L, D), lambda: (0, 0)),
            pl.BlockSpec((2 * D, H), lambda: (0, 0)),
            pl.BlockSpec((2 * D, H), lambda: (0, 0)),
            pl.BlockSpec((1, H), lambda: (0, 0)),
        ],
        out_specs=[
            pl.BlockSpec((L, H), lambda: (0, 0)),
            pl.BlockSpec((L, H), lambda: (0, 0)),
        ],
        out_shape=[
            jax.ShapeDtypeStruct((L, H), jnp.float32),
            jax.ShapeDtypeStruct((L, H), jnp.float32),
        ],
    )(pos, W1_imp, W1_enc, b1e)

    grid = (B // TB,)
    out = pl.pallas_call(
        _pool_kernel,
        grid=grid,
        in_specs=[
            pl.BlockSpec((TB, L, D), lambda i: (i, 0, 0)),
            pl.BlockSpec((TB, L), lambda i: (i, 0)),
            pl.BlockSpec((L, H), lambda i: (0, 0)),
            pl.BlockSpec((L, H), lambda i: (0, 0)),
            pl.BlockSpec((2 * D, H), lambda i: (0, 0)),
            pl.BlockSpec((1, H), lambda i: (0, 0)),
            pl.BlockSpec((H, 1), lambda i: (0, 0)),
            pl.BlockSpec((2 * D, H), lambda i: (0, 0)),
            pl.BlockSpec((H, O), lambda i: (0, 0)),
            pl.BlockSpec((1, O), lambda i: (0, 0)),
        ],
        out_specs=pl.BlockSpec((TB, O), lambda i: (i, 0)),
        out_shape=jax.ShapeDtypeStruct((B, O), jnp.float32),
        compiler_params=pltpu.CompilerParams(
            dimension_semantics=("parallel",),
        ),
    )(sequence_emb, mask, p_imp, p_enc, W1_imp, b1i, w2i, W1_enc, W2_enc, b2e)
    return out


# speedup vs baseline: 1.6154x; 1.0111x over previous
import jax
import jax.numpy as jnp
from jax.experimental import pallas as pl
from jax.experimental.pallas import tpu as pltpu

B, L, D = 1024, 200, 128
H, O = 512, 128
TOP_K = 50
TB = 128
TH = 8
NSUB = TB // TH


def _pos_kernel(pos_ref, w1i_ref, w1e_ref, b1e_ref, p_imp_ref, p_enc_ref):
    pos = pos_ref[...]
    p_imp_ref[...] = jnp.dot(
        pos, w1i_ref[D:, :], preferred_element_type=jnp.float32)
    p_enc_ref[...] = (
        jnp.dot(pos, w1e_ref[D:, :], preferred_element_type=jnp.float32)
        + b1e_ref[...])


def _pool_kernel(seq_ref, mask_ref, p_imp_ref, p_enc_ref, w1i_ref, b1i_ref,
                 w2i_ref, w1e_ref, w2e_ref, b2e_ref, out_ref):
    dn = (((2,), (0,)), ((), ()))
    p_imp = (p_imp_ref[...] + b1i_ref[...])[None, :, :]
    p_enc = p_enc_ref[...][None, :, :]

    def hidden(half):
        seq = seq_ref[pl.ds(half * TH, TH), :, :]
        a = jax.lax.dot_general(seq, w1i_ref[:D, :], dn,
                                preferred_element_type=jnp.float32)
        a2 = jax.lax.dot_general(seq, w1e_ref[:D, :], dn,
                                 preferred_element_type=jnp.float32)
        h = jnp.maximum(a + p_imp, 0.0)
        return h, a2

    def score(h, half):
        s = jnp.dot(h.reshape(TH * L, H), w2i_ref[...],
                    preferred_element_type=jnp.float32).reshape(TH, L)
        m = mask_ref[pl.ds(half * TH, TH), :]
        return jnp.where(m == 0, jnp.float32(-1e9), s)

    def pool(scores, a2, half):
        s_i = scores[:, :, None]
        s_j = scores[:, None, :]
        ii = jax.lax.broadcasted_iota(jnp.int32, (TH, L, L), 1)
        jj = jax.lax.broadcasted_iota(jnp.int32, (TH, L, L), 2)
        beats = (s_j > s_i) | ((s_j == s_i) & (jj < ii))
        rank = jnp.sum(jnp.where(beats, 1.0, 0.0), axis=2)
        selw = (rank < TOP_K).astype(jnp.float32)

        h2 = jnp.maximum(a2 + p_enc, 0.0)
        pooled = jnp.sum(h2 * selw[:, :, None], axis=1) * (1.0 / TOP_K)
        out_ref[pl.ds(half * TH, TH), :] = (
            jnp.dot(pooled, w2e_ref[...], preferred_element_type=jnp.float32)
            + b2e_ref[...])

    h_prev, a2_prev = hidden(0)
    s_prev = score(h_prev, 0)
    for half in range(1, NSUB):
        h_cur, a2_cur = hidden(half)
        s_cur = score(h_cur, half)
        pool(s_prev, a2_prev, half - 1)
        s_prev, a2_prev = s_cur, a2_cur
    pool(s_prev, a2_prev, NSUB - 1)


@jax.jit
def kernel(sequence_emb, mask, pos_table, W1_imp, b1_imp, W2_imp, b2_imp,
           W1_enc, b1_enc, W2_enc, b2_enc):
    del b2_imp
    pos = pos_table[:L]
    b1i = b1_imp.reshape(1, H)
    b1e = b1_enc.reshape(1, H)
    w2i = W2_imp.reshape(H, 1)
    b2e = b2_enc.reshape(1, O)

    p_imp, p_enc = pl.pallas_call(
        _pos_kernel,
        in_specs=[
            pl.BlockSpec((L, D), lambda: (0, 0)),
            pl.BlockSpec((2 * D, H), lambda: (0, 0)),
            pl.BlockSpec((2 * D, H), lambda: (0, 0)),
            pl.BlockSpec((1, H), lambda: (0, 0)),
        ],
        out_specs=[
            pl.BlockSpec((L, H), lambda: (0, 0)),
            pl.BlockSpec((L, H), lambda: (0, 0)),
        ],
        out_shape=[
            jax.ShapeDtypeStruct((L, H), jnp.float32),
            jax.ShapeDtypeStruct((L, H), jnp.float32),
        ],
    )(pos, W1_imp, W1_enc, b1e)

    grid = (B // TB,)
    out = pl.pallas_call(
        _pool_kernel,
        grid=grid,
        in_specs=[
            pl.BlockSpec((TB, L, D), lambda i: (i, 0, 0)),
            pl.BlockSpec((TB, L), lambda i: (i, 0)),
            pl.BlockSpec((L, H), lambda i: (0, 0)),
            pl.BlockSpec((L, H), lambda i: (0, 0)),
            pl.BlockSpec((2 * D, H), lambda i: (0, 0)),
            pl.BlockSpec((1, H), lambda i: (0, 0)),
            pl.BlockSpec((H, 1), lambda i: (0, 0)),
            pl.BlockSpec((2 * D, H), lambda i: (0, 0)),
            pl.BlockSpec((H, O), lambda i: (0, 0)),
            pl.BlockSpec((1, O), lambda i: (0, 0)),
        ],
        out_specs=pl.BlockSpec((TB, O), lambda i: (i, 0)),
        out_shape=jax.ShapeDtypeStruct((B, O), jnp.float32),
        compiler_params=pltpu.CompilerParams(
            dimension_semantics=("parallel",),
        ),
    )(sequence_emb, mask, p_imp, p_enc, W1_imp, b1i, w2i, W1_enc, W2_enc, b2e)
    return out


# speedup vs baseline: 1.6801x; 1.0401x over previous
import jax
import jax.numpy as jnp
from jax.experimental import pallas as pl
from jax.experimental.pallas import tpu as pltpu

B, L, D = 1024, 200, 128
H, O = 512, 128
TOP_K = 50
TB = 128
TH = 16
NSUB = TB // TH


def _pos_kernel(pos_ref, w1i_ref, w1e_ref, b1e_ref, p_imp_ref, p_enc_ref):
    pos = pos_ref[...]
    p_imp_ref[...] = jnp.dot(
        pos, w1i_ref[D:, :], preferred_element_type=jnp.float32)
    p_enc_ref[...] = (
        jnp.dot(pos, w1e_ref[D:, :], preferred_element_type=jnp.float32)
        + b1e_ref[...])


def _pool_kernel(seq_ref, mask_ref, p_imp_ref, p_enc_ref, w1i_ref, b1i_ref,
                 w2i_ref, w1e_ref, w2e_ref, b2e_ref, out_ref):
    dn = (((2,), (0,)), ((), ()))
    p_imp = (p_imp_ref[...] + b1i_ref[...])[None, :, :]
    p_enc = p_enc_ref[...][None, :, :]

    def hidden(half):
        seq = seq_ref[pl.ds(half * TH, TH), :, :]
        a = jax.lax.dot_general(seq, w1i_ref[:D, :], dn,
                                preferred_element_type=jnp.float32)
        a2 = jax.lax.dot_general(seq, w1e_ref[:D, :], dn,
                                 preferred_element_type=jnp.float32)
        h = jnp.maximum(a + p_imp, 0.0)
        return h, a2

    def score(h, half):
        s = jnp.dot(h.reshape(TH * L, H), w2i_ref[...],
                    preferred_element_type=jnp.float32).reshape(TH, L)
        m = mask_ref[pl.ds(half * TH, TH), :]
        return jnp.where(m == 0, jnp.float32(-1e9), s)

    def pool(scores, a2, half):
        s_i = scores[:, :, None]
        s_j = scores[:, None, :]
        ii = jax.lax.broadcasted_iota(jnp.int32, (TH, L, L), 1)
        jj = jax.lax.broadcasted_iota(jnp.int32, (TH, L, L), 2)
        beats = (s_j > s_i) | ((s_j == s_i) & (jj < ii))
        rank = jnp.sum(jnp.where(beats, 1.0, 0.0), axis=2)
        selw = (rank < TOP_K).astype(jnp.float32)

        h2 = jnp.maximum(a2 + p_enc, 0.0)
        pooled = jnp.sum(h2 * selw[:, :, None], axis=1) * (1.0 / TOP_K)
        out_ref[pl.ds(half * TH, TH), :] = (
            jnp.dot(pooled, w2e_ref[...], preferred_element_type=jnp.float32)
            + b2e_ref[...])

    h_prev, a2_prev = hidden(0)
    s_prev = score(h_prev, 0)
    for half in range(1, NSUB):
        h_cur, a2_cur = hidden(half)
        s_cur = score(h_cur, half)
        pool(s_prev, a2_prev, half - 1)
        s_prev, a2_prev = s_cur, a2_cur
    pool(s_prev, a2_prev, NSUB - 1)


@jax.jit
def kernel(sequence_emb, mask, pos_table, W1_imp, b1_imp, W2_imp, b2_imp,
           W1_enc, b1_enc, W2_enc, b2_enc):
    del b2_imp
    pos = pos_table[:L]
    b1i = b1_imp.reshape(1, H)
    b1e = b1_enc.reshape(1, H)
    w2i = W2_imp.reshape(H, 1)
    b2e = b2_enc.reshape(1, O)

    p_imp, p_enc = pl.pallas_call(
        _pos_kernel,
        in_specs=[
            pl.BlockSpec((L, D), lambda: (0, 0)),
            pl.BlockSpec((2 * D, H), lambda: (0, 0)),
            pl.BlockSpec((2 * D, H), lambda: (0, 0)),
            pl.BlockSpec((1, H), lambda: (0, 0)),
        ],
        out_specs=[
            pl.BlockSpec((L, H), lambda: (0, 0)),
            pl.BlockSpec((L, H), lambda: (0, 0)),
        ],
        out_shape=[
            jax.ShapeDtypeStruct((L, H), jnp.float32),
            jax.ShapeDtypeStruct((L, H), jnp.float32),
        ],
    )(pos, W1_imp, W1_enc, b1e)

    grid = (B // TB,)
    out = pl.pallas_call(
        _pool_kernel,
        grid=grid,
        in_specs=[
            pl.BlockSpec((TB, L, D), lambda i: (i, 0, 0)),
            pl.BlockSpec((TB, L), lambda i: (i, 0)),
            pl.BlockSpec((L, H), lambda i: (0, 0)),
            pl.BlockSpec((L, H), lambda i: (0, 0)),
            pl.BlockSpec((2 * D, H), lambda i: (0, 0)),
            pl.BlockSpec((1, H), lambda i: (0, 0)),
            pl.BlockSpec((H, 1), lambda i: (0, 0)),
            pl.BlockSpec((2 * D, H), lambda i: (0, 0)),
            pl.BlockSpec((H, O), lambda i: (0, 0)),
            pl.BlockSpec((1, O), lambda i: (0, 0)),
        ],
        out_specs=pl.BlockSpec((TB, O), lambda i: (i, 0)),
        out_shape=jax.ShapeDtypeStruct((B, O), jnp.float32),
        compiler_params=pltpu.CompilerParams(
            dimension_semantics=("parallel",),
        ),
    )(sequence_emb, mask, p_imp, p_enc, W1_imp, b1i, w2i, W1_enc, W2_enc, b2e)
    return out
